# Initial kernel scaffold; baseline (speedup 1.0000x reference)
import jax
import jax.numpy as jnp
from jax.experimental import pallas as pl


def kernel(codes, pred_class, centroids):
    raise NotImplementedError("write your pallas kernel here")


# speedup vs baseline: 1.8448x; 1.8448x over previous
import functools

import jax
import jax.numpy as jnp
from jax import lax
from jax.experimental import pallas as pl
from jax.experimental.pallas import tpu as pltpu
from jax.experimental.pallas import tpu_sc as plsc

NUM_CLASSES = 8192
CODE_DIM = 64
K = 4
KD = K * CODE_DIM
CHUNK = 128


def kernel(codes, pred_class, centroids):
    B = codes.shape[0]
    NC, NS = 2, 16
    NW = NC * NS
    rows_per_w = B // NW
    n_chunks = rows_per_w // CHUNK
    assert rows_per_w * NW == B and n_chunks * CHUNK == rows_per_w

    table = centroids.reshape(NUM_CLASSES, KD)
    codes_flat = codes.reshape(B * CODE_DIM)
    mesh = plsc.VectorSubcoreMesh(core_axis_name="c", subcore_axis_name="s")

    @functools.partial(
        pl.kernel,
        mesh=mesh,
        out_type=jax.ShapeDtypeStruct((B,), jnp.float32),
        compiler_params=pltpu.CompilerParams(needs_layout_passes=False),
        scratch_types=[
            pltpu.VMEM((CHUNK,), jnp.int32),
            pltpu.VMEM((CHUNK, KD), jnp.float32),
            pltpu.VMEM((CHUNK * CODE_DIM,), jnp.float32),
            pltpu.VMEM((CHUNK,), jnp.float32),
            pltpu.SemaphoreType.DMA,
        ],
    )
    def sc_kernel(codes_hbm, idx_hbm, table_hbm, out_hbm,
                  idx_v, rows_v, codes_v, out_v, sem):
        wid = lax.axis_index("s") * NC + lax.axis_index("c")
        lane = lax.iota(jnp.int32, 16)

        def chunk_body(ci, carry):
            base = wid * rows_per_w + ci * CHUNK
            pltpu.sync_copy(idx_hbm.at[pl.ds(base, CHUNK)], idx_v)
            gather = pltpu.async_copy(table_hbm.at[idx_v], rows_v, sem)
            pltpu.sync_copy(
                codes_hbm.at[pl.ds(base * CODE_DIM, CHUNK * CODE_DIM)], codes_v)
            gather.wait()

            def group_body(g, c2):
                rows16 = lane + g * 16
                crow = rows16 * CODE_DIM
                acc = [jnp.zeros((16,), jnp.float32) for _ in range(K)]
                for d in range(CODE_DIM):
                    cv = plsc.load_gather(codes_v, [crow + d])
                    for k in range(K):
                        tv = plsc.load_gather(
                            rows_v,
                            [rows16, jnp.full((16,), k * CODE_DIM + d, jnp.int32)])
                        acc[k] = acc[k] + jnp.abs(cv - tv)
                m = jnp.minimum(jnp.minimum(acc[0], acc[1]),
                                jnp.minimum(acc[2], acc[3]))
                out_v[pl.ds(g * 16, 16)] = m * (1.0 / CODE_DIM)
                return c2

            lax.fori_loop(0, CHUNK // 16, group_body, 0)
            pltpu.sync_copy(out_v, out_hbm.at[pl.ds(base, CHUNK)])
            return carry

        lax.fori_loop(0, n_chunks, chunk_body, 0)

    return sc_kernel(codes_flat, pred_class, table)


# speedup vs baseline: 1.9541x; 1.0593x over previous
import functools

import jax
import jax.numpy as jnp
from jax import lax
from jax.experimental import pallas as pl
from jax.experimental.pallas import tpu as pltpu
from jax.experimental.pallas import tpu_sc as plsc

NUM_CLASSES = 8192
CODE_DIM = 64
K = 4
KD = K * CODE_DIM
CHUNK = 128


def kernel(codes, pred_class, centroids):
    B = codes.shape[0]
    info = plsc.get_sparse_core_info()
    NC, NS = info.num_cores, info.num_subcores
    NW = NC * NS
    rows_per_w = B // NW
    n_chunks = rows_per_w // CHUNK
    assert rows_per_w * NW == B and n_chunks * CHUNK == rows_per_w
    assert n_chunks % 2 == 0

    table = centroids.reshape(NUM_CLASSES, KD)
    codes_flat = codes.reshape(B * CODE_DIM)
    mesh = plsc.VectorSubcoreMesh(core_axis_name="c", subcore_axis_name="s")

    @functools.partial(
        pl.kernel,
        mesh=mesh,
        out_type=jax.ShapeDtypeStruct((B,), jnp.float32),
        compiler_params=pltpu.CompilerParams(needs_layout_passes=False),
        scratch_types=[
            pltpu.VMEM((CHUNK,), jnp.int32),
            pltpu.VMEM((CHUNK,), jnp.int32),
            pltpu.VMEM((CHUNK, KD), jnp.float32),
            pltpu.VMEM((CHUNK, KD), jnp.float32),
            pltpu.VMEM((CHUNK * CODE_DIM,), jnp.float32),
            pltpu.VMEM((CHUNK * CODE_DIM,), jnp.float32),
            pltpu.VMEM((CHUNK,), jnp.float32),
            pltpu.SemaphoreType.DMA,
            pltpu.SemaphoreType.DMA,
            pltpu.SemaphoreType.DMA,
            pltpu.SemaphoreType.DMA,
        ],
    )
    def sc_kernel(codes_hbm, idx_hbm, table_hbm, out_hbm,
                  idx0, idx1, rows0, rows1, cod0, cod1, out_v,
                  sg0, sg1, sc0, sc1):
        idx_v = (idx0, idx1)
        rows_v = (rows0, rows1)
        codes_v = (cod0, cod1)
        sem_g = (sg0, sg1)
        sem_c = (sc0, sc1)

        wid = lax.axis_index("s") * NC + lax.axis_index("c")
        lane = lax.iota(jnp.int32, 16)

        def stage(b, ci):
            base = wid * rows_per_w + ci * CHUNK
            pltpu.sync_copy(idx_hbm.at[pl.ds(base, CHUNK)], idx_v[b])
            pltpu.async_copy(table_hbm.at[idx_v[b]], rows_v[b], sem_g[b])
            pltpu.async_copy(
                codes_hbm.at[pl.ds(base * CODE_DIM, CHUNK * CODE_DIM)],
                codes_v[b], sem_c[b])

        def wait(b):
            pltpu.make_async_copy(
                table_hbm.at[idx_v[b]], rows_v[b], sem_g[b]).wait()
            pltpu.make_async_copy(
                codes_hbm.at[pl.ds(0, CHUNK * CODE_DIM)],
                codes_v[b], sem_c[b]).wait()

        def compute(b, ci):
            base = wid * rows_per_w + ci * CHUNK

            def group_body(g, c2):
                rows16 = lane + g * 16
                crow = rows16 * CODE_DIM
                acc = [jnp.zeros((16,), jnp.float32) for _ in range(K)]
                for d in range(CODE_DIM):
                    cv = plsc.load_gather(codes_v[b], [crow + d])
                    for k in range(K):
                        tv = plsc.load_gather(
                            rows_v[b],
                            [rows16, jnp.full((16,), k * CODE_DIM + d, jnp.int32)])
                        acc[k] = acc[k] + jnp.abs(cv - tv)
                m = jnp.minimum(jnp.minimum(acc[0], acc[1]),
                                jnp.minimum(acc[2], acc[3]))
                out_v[pl.ds(g * 16, 16)] = m * (1.0 / CODE_DIM)
                return c2

            lax.fori_loop(0, CHUNK // 16, group_body, 0)
            pltpu.sync_copy(out_v, out_hbm.at[pl.ds(base, CHUNK)])

        stage(0, 0)

        def outer(cc, carry):
            for b in range(2):
                ci = cc * 2 + b

                @pl.when(ci + 1 < n_chunks)
                def _():
                    stage(1 - b, ci + 1)

                wait(b)
                compute(b, ci)
            return carry

        lax.fori_loop(0, n_chunks // 2, outer, 0)

    return sc_kernel(codes_flat, pred_class, table)


# speedup vs baseline: 9.8067x; 5.0185x over previous
import functools

import jax
import jax.numpy as jnp
from jax import lax
from jax.experimental import pallas as pl
from jax.experimental.pallas import tpu as pltpu
from jax.experimental.pallas import tpu_sc as plsc

NUM_CLASSES = 8192
CODE_DIM = 64
K = 4
KD = K * CODE_DIM
NSLICE = CODE_DIM // 16
CHUNK = 128
GROUP = 16


def kernel(codes, pred_class, centroids):
    B = codes.shape[0]
    NC, NS = 2, 16
    NW = NC * NS
    rows_per_w = B // NW
    n_chunks = rows_per_w // CHUNK
    assert rows_per_w * NW == B and n_chunks * CHUNK == rows_per_w
    assert n_chunks % 2 == 0

    table = centroids.reshape(NUM_CLASSES, KD)
    mesh = plsc.VectorSubcoreMesh(core_axis_name="c", subcore_axis_name="s")

    @functools.partial(
        pl.kernel,
        mesh=mesh,
        out_type=jax.ShapeDtypeStruct((B,), jnp.float32),
        compiler_params=pltpu.CompilerParams(needs_layout_passes=False),
        scratch_types=[
            pltpu.VMEM((CHUNK,), jnp.int32),
            pltpu.VMEM((CHUNK,), jnp.int32),
            pltpu.VMEM((CHUNK, KD), jnp.float32),
            pltpu.VMEM((CHUNK, KD), jnp.float32),
            pltpu.VMEM((CHUNK, CODE_DIM), jnp.float32),
            pltpu.VMEM((CHUNK, CODE_DIM), jnp.float32),
            pltpu.VMEM((CHUNK,), jnp.float32),
            pltpu.SemaphoreType.DMA,
            pltpu.SemaphoreType.DMA,
            pltpu.SemaphoreType.DMA,
            pltpu.SemaphoreType.DMA,
        ],
    )
    def sc_kernel(codes_hbm, idx_hbm, table_hbm, out_hbm,
                  idx0, idx1, rows0, rows1, cod0, cod1, out_v,
                  sg0, sg1, sc0, sc1):
        idx_v = (idx0, idx1)
        rows_v = (rows0, rows1)
        codes_v = (cod0, cod1)
        sem_g = (sg0, sg1)
        sem_c = (sc0, sc1)

        wid = lax.axis_index("s") * NC + lax.axis_index("c")
        lane = lax.iota(jnp.int32, 16)
        lane_eq = [lane == j for j in range(GROUP)]

        def stage(b, ci):
            base = wid * rows_per_w + ci * CHUNK
            pltpu.sync_copy(idx_hbm.at[pl.ds(base, CHUNK)], idx_v[b])
            pltpu.async_copy(table_hbm.at[idx_v[b]], rows_v[b], sem_g[b])
            pltpu.async_copy(codes_hbm.at[pl.ds(base, CHUNK)],
                             codes_v[b], sem_c[b])

        def wait(b):
            pltpu.make_async_copy(
                table_hbm.at[idx_v[b]], rows_v[b], sem_g[b]).wait()
            pltpu.make_async_copy(
                codes_hbm.at[pl.ds(0, CHUNK)], codes_v[b], sem_c[b]).wait()

        def compute(b, ci):
            base = wid * rows_per_w + ci * CHUNK

            def group_body(g, c2):
                res = jnp.zeros((16,), jnp.float32)
                for r16 in range(GROUP):
                    r = g * GROUP + r16
                    c = [codes_v[b][r, pl.ds(16 * j, 16)]
                         for j in range(NSLICE)]
                    best = None
                    for k in range(K):
                        s = jnp.zeros((16,), jnp.float32)
                        for j in range(NSLICE):
                            t = rows_v[b][r, pl.ds(k * CODE_DIM + 16 * j, 16)]
                            s = s + jnp.abs(c[j] - t)
                        tot = jnp.sum(s)
                        best = tot if best is None else jnp.minimum(best, tot)
                    res = jnp.where(lane_eq[r16],
                                    jnp.full((16,), best * (1.0 / CODE_DIM)),
                                    res)
                out_v[pl.ds(g * GROUP, GROUP)] = res
                return c2

            lax.fori_loop(0, CHUNK // GROUP, group_body, 0)
            pltpu.sync_copy(out_v, out_hbm.at[pl.ds(base, CHUNK)])

        stage(0, 0)

        def outer(cc, carry):
            for b in range(2):
                ci = cc * 2 + b

                @pl.when(ci + 1 < n_chunks)
                def _():
                    stage(1 - b, ci + 1)

                wait(b)
                compute(b, ci)
            return carry

        lax.fori_loop(0, n_chunks // 2, outer, 0)

    return sc_kernel(codes, pred_class, table)
